# baseline (speedup 1.0000x reference)
import functools

import jax
import jax.numpy as jnp
from jax import lax
from jax.experimental import pallas as pl
from jax.experimental.pallas import tpu as pltpu
from jax.experimental.pallas import tpu_sc as plsc

BATCH = 16384
EMB = 64
NC = 2
NS = 16
NW = NC * NS
B_PER_W = BATCH // NW


def _gather_body(tableT_hbm, idx_hbm, outT_hbm, idx_v, rows_v, sem):
    wid = lax.axis_index("s") * NC + lax.axis_index("c")
    base = wid * B_PER_W
    pltpu.sync_copy(idx_hbm.at[pl.ds(base, B_PER_W)], idx_v)

    @pl.loop(0, EMB)
    def _(c):
        pltpu.make_async_copy(
            tableT_hbm.at[c].at[idx_v], rows_v.at[c], sem
        ).start()

    @pl.loop(0, EMB)
    def _(c):
        pltpu.make_async_copy(
            tableT_hbm.at[c].at[idx_v], rows_v.at[c], sem
        ).wait()

    pltpu.sync_copy(rows_v, outT_hbm.at[:, pl.ds(base, B_PER_W)])


def _sc_gather(tableT, idx):
    mesh = plsc.VectorSubcoreMesh(core_axis_name="c", subcore_axis_name="s")
    k = pl.kernel(
        _gather_body,
        mesh=mesh,
        out_type=jax.ShapeDtypeStruct((EMB, BATCH), jnp.float32),
        scratch_types=[
            pltpu.VMEM((B_PER_W,), jnp.int32),
            pltpu.VMEM((EMB, B_PER_W), jnp.float32),
            pltpu.SemaphoreType.DMA,
        ],
        compiler_params=pltpu.CompilerParams(use_tc_tiling_on_sc=False),
    )
    return k(tableT, idx)


MLP_BLK = 2048


def _mlp_body(stats_ref, embT_ref, age_ref, msin_ref, mcos_ref,
              w1a_ref, w1bT_ref, b1_ref, w2_ref, b2_ref, outT_ref):
    mean = stats_ref[0, 0]
    inv = lax.rsqrt(stats_ref[0, 1] + 1e-7)
    dn = (((0,), (0,)), ((), ()))
    hT = lax.dot_general(w1a_ref[...], embT_ref[...], dn,
                         preferred_element_type=jnp.float32)
    hT += w1bT_ref[:, 0:1] * ((age_ref[...] - mean) * inv)
    hT += w1bT_ref[:, 1:2] * msin_ref[...]
    hT += w1bT_ref[:, 2:3] * mcos_ref[...]
    hT = jnp.maximum(hT + b1_ref[...], 0.0)
    outT_ref[...] = lax.dot_general(
        w2_ref[...], hT, dn, preferred_element_type=jnp.float32
    ) + b2_ref[...]


def _tc_mlp(embT, age, msin, mcos, stats, w1a, w1bT, b1, w2, b2):
    grid = (BATCH // MLP_BLK,)
    return pl.pallas_call(
        _mlp_body,
        grid=grid,
        in_specs=[
            pl.BlockSpec(memory_space=pltpu.SMEM),
            pl.BlockSpec((EMB, MLP_BLK), lambda i: (0, i)),
            pl.BlockSpec((1, MLP_BLK), lambda i: (0, i)),
            pl.BlockSpec((1, MLP_BLK), lambda i: (0, i)),
            pl.BlockSpec((1, MLP_BLK), lambda i: (0, i)),
            pl.BlockSpec((EMB, EMB), lambda i: (0, 0)),
            pl.BlockSpec((EMB, 3), lambda i: (0, 0)),
            pl.BlockSpec((EMB, 1), lambda i: (0, 0)),
            pl.BlockSpec((EMB, EMB), lambda i: (0, 0)),
            pl.BlockSpec((EMB, 1), lambda i: (0, 0)),
        ],
        out_specs=pl.BlockSpec((EMB, MLP_BLK), lambda i: (0, i)),
        out_shape=jax.ShapeDtypeStruct((EMB, BATCH), jnp.float32),
    )(stats, embT, age, msin, mcos, w1a, w1bT, b1, w2, b2)


def kernel(customer_id, age, month_sin, month_cos, table, age_mean, age_var,
           W1, b1, W2, b2):
    idx = customer_id.astype(jnp.int32)
    tableT = table.T
    embT = _sc_gather(tableT, idx)
    stats = jnp.stack([age_mean, age_var]).reshape(1, 2)
    outT = _tc_mlp(
        embT,
        age.reshape(1, BATCH),
        month_sin.reshape(1, BATCH),
        month_cos.reshape(1, BATCH),
        stats,
        W1[:EMB],
        W1[EMB:].T,
        b1.reshape(EMB, 1),
        W2,
        b2.reshape(EMB, 1),
    )
    return outT.T


# speedup vs baseline: 5.7622x; 5.7622x over previous
import jax
import jax.numpy as jnp
from jax import lax
from jax.experimental import pallas as pl
from jax.experimental.pallas import tpu as pltpu
from jax.experimental.pallas import tpu_sc as plsc

BATCH = 16384
EMB = 64
VROWS = 1000001
NC = 2
NS = 16
NW = NC * NS
B_PER_W = BATCH // NW

TR_BLK = 2048


def _tr_body(src_ref, dst_ref):
    dst_ref[...] = src_ref[...].T


def _tc_transpose(tableT):
    grid = (pl.cdiv(VROWS, TR_BLK),)
    return pl.pallas_call(
        _tr_body,
        grid=grid,
        in_specs=[pl.BlockSpec((EMB, TR_BLK), lambda i: (0, i))],
        out_specs=pl.BlockSpec((TR_BLK, EMB), lambda i: (i, 0)),
        out_shape=jax.ShapeDtypeStruct((VROWS, EMB), jnp.float32),
    )(tableT)


def _gather_body(table_hbm, idx_hbm, out_hbm, idx_v, rows_v, sem):
    wid = lax.axis_index("s") * NC + lax.axis_index("c")
    base = wid * B_PER_W
    pltpu.sync_copy(idx_hbm.at[pl.ds(base, B_PER_W)], idx_v)
    pltpu.async_copy(table_hbm.at[idx_v], rows_v, sem).wait()
    pltpu.sync_copy(rows_v, out_hbm.at[pl.ds(base, B_PER_W)])


def _sc_gather(table, idx):
    mesh = plsc.VectorSubcoreMesh(core_axis_name="c", subcore_axis_name="s")
    k = pl.kernel(
        _gather_body,
        mesh=mesh,
        out_type=jax.ShapeDtypeStruct((BATCH, EMB), jnp.float32),
        scratch_types=[
            pltpu.VMEM((B_PER_W,), jnp.int32),
            pltpu.VMEM((B_PER_W, EMB), jnp.float32),
            pltpu.SemaphoreType.DMA,
        ],
        compiler_params=pltpu.CompilerParams(use_tc_tiling_on_sc=False),
    )
    return k(table, idx)


MLP_BLK = 2048


def _mlp_body(stats_ref, emb_ref, age_ref, msin_ref, mcos_ref,
              w1a_ref, w1b_ref, b1_ref, w2_ref, b2_ref, out_ref):
    mean = stats_ref[0, 0]
    inv = lax.rsqrt(stats_ref[0, 1] + 1e-7)
    h = jnp.dot(emb_ref[...], w1a_ref[...], preferred_element_type=jnp.float32)
    h += ((age_ref[...] - mean) * inv) * w1b_ref[0:1, :]
    h += msin_ref[...] * w1b_ref[1:2, :]
    h += mcos_ref[...] * w1b_ref[2:3, :]
    h = jnp.maximum(h + b1_ref[...], 0.0)
    out_ref[...] = jnp.dot(h, w2_ref[...], preferred_element_type=jnp.float32) + b2_ref[...]


def _tc_mlp(emb, age, msin, mcos, stats, w1a, w1b, b1, w2, b2):
    grid = (BATCH // MLP_BLK,)
    return pl.pallas_call(
        _mlp_body,
        grid=grid,
        in_specs=[
            pl.BlockSpec(memory_space=pltpu.SMEM),
            pl.BlockSpec((MLP_BLK, EMB), lambda i: (i, 0)),
            pl.BlockSpec((MLP_BLK, 1), lambda i: (i, 0)),
            pl.BlockSpec((MLP_BLK, 1), lambda i: (i, 0)),
            pl.BlockSpec((MLP_BLK, 1), lambda i: (i, 0)),
            pl.BlockSpec((EMB, EMB), lambda i: (0, 0)),
            pl.BlockSpec((3, EMB), lambda i: (0, 0)),
            pl.BlockSpec((1, EMB), lambda i: (0, 0)),
            pl.BlockSpec((EMB, EMB), lambda i: (0, 0)),
            pl.BlockSpec((1, EMB), lambda i: (0, 0)),
        ],
        out_specs=pl.BlockSpec((MLP_BLK, EMB), lambda i: (i, 0)),
        out_shape=jax.ShapeDtypeStruct((BATCH, EMB), jnp.float32),
    )(stats, emb, age, msin, mcos, w1a, w1b, b1, w2, b2)


def kernel(customer_id, age, month_sin, month_cos, table, age_mean, age_var,
           W1, b1, W2, b2):
    idx = customer_id.astype(jnp.int32)
    tableR = _tc_transpose(table.T)
    emb = _sc_gather(tableR, idx)
    stats = jnp.stack([age_mean, age_var]).reshape(1, 2)
    return _tc_mlp(
        emb,
        age.reshape(-1, 1),
        month_sin.reshape(-1, 1),
        month_cos.reshape(-1, 1),
        stats,
        W1[:EMB],
        W1[EMB:],
        b1.reshape(1, EMB),
        W2,
        b2.reshape(1, EMB),
    )


# speedup vs baseline: 16.1032x; 2.7946x over previous
import jax
import jax.numpy as jnp
from jax import lax
from jax.experimental import pallas as pl
from jax.experimental.pallas import tpu as pltpu
from jax.experimental.pallas import tpu_sc as plsc

BATCH = 16384
EMB = 64
VROWS = 1000001
NC = 2
NS = 16
NW = NC * NS
B_PER_W = BATCH // NW

TR_BLK = 8192
TR_STEPS = pl.cdiv(VROWS, TR_BLK)
PAIR_ROWS = TR_STEPS * (TR_BLK // 2)


HALF = TR_BLK // 2


def _pair_body(src_ref, dst_ref):
    dst_ref[:, :EMB] = src_ref[:, :HALF].T
    dst_ref[:, EMB:] = src_ref[:, HALF:].T


def _tc_pair_relayout(tableT):
    return pl.pallas_call(
        _pair_body,
        grid=(TR_STEPS,),
        in_specs=[pl.BlockSpec((EMB, TR_BLK), lambda i: (0, i))],
        out_specs=pl.BlockSpec((TR_BLK // 2, 2 * EMB), lambda i: (i, 0)),
        out_shape=jax.ShapeDtypeStruct((PAIR_ROWS, 2 * EMB), jnp.float32),
    )(tableT)


def _gather_body(pairs_hbm, pidx_hbm, out_hbm, idx_v, rows_v, sem):
    wid = lax.axis_index("s") * NC + lax.axis_index("c")
    base = wid * B_PER_W
    pltpu.sync_copy(pidx_hbm.at[pl.ds(base, B_PER_W)], idx_v)
    pltpu.async_copy(pairs_hbm.at[idx_v], rows_v, sem).wait()
    pltpu.sync_copy(rows_v, out_hbm.at[pl.ds(base, B_PER_W)])


def _sc_gather(pairs, pidx):
    mesh = plsc.VectorSubcoreMesh(core_axis_name="c", subcore_axis_name="s")
    k = pl.kernel(
        _gather_body,
        mesh=mesh,
        out_type=jax.ShapeDtypeStruct((BATCH, 2 * EMB), jnp.float32),
        scratch_types=[
            pltpu.VMEM((B_PER_W,), jnp.int32),
            pltpu.VMEM((B_PER_W, 2 * EMB), jnp.float32),
            pltpu.SemaphoreType.DMA,
        ],
        compiler_params=pltpu.CompilerParams(use_tc_tiling_on_sc=False),
    )
    return k(pairs, pidx)


MLP_BLK = 2048


def _mlp_body(stats_ref, emb2_ref, idx_ref, age_ref, msin_ref, mcos_ref,
              w1a_ref, w1b_ref, b1_ref, w2_ref, b2_ref, out_ref):
    mean = stats_ref[0, 0]
    inv = lax.rsqrt(stats_ref[0, 1] + 1e-7)
    emb2 = emb2_ref[...]
    upper = lax.bitwise_and(
        lax.shift_right_logical(idx_ref[...], 12), 1
    ) == 1
    emb = jnp.where(upper, emb2[:, EMB:], emb2[:, :EMB])
    h = jnp.dot(emb, w1a_ref[...], preferred_element_type=jnp.float32)
    h += ((age_ref[...] - mean) * inv) * w1b_ref[0:1, :]
    h += msin_ref[...] * w1b_ref[1:2, :]
    h += mcos_ref[...] * w1b_ref[2:3, :]
    h = jnp.maximum(h + b1_ref[...], 0.0)
    out_ref[...] = jnp.dot(h, w2_ref[...], preferred_element_type=jnp.float32) + b2_ref[...]


def _tc_mlp(emb2, idx, age, msin, mcos, stats, w1a, w1b, b1, w2, b2):
    grid = (BATCH // MLP_BLK,)
    return pl.pallas_call(
        _mlp_body,
        grid=grid,
        in_specs=[
            pl.BlockSpec(memory_space=pltpu.SMEM),
            pl.BlockSpec((MLP_BLK, 2 * EMB), lambda i: (i, 0)),
            pl.BlockSpec((MLP_BLK, 1), lambda i: (i, 0)),
            pl.BlockSpec((MLP_BLK, 1), lambda i: (i, 0)),
            pl.BlockSpec((MLP_BLK, 1), lambda i: (i, 0)),
            pl.BlockSpec((MLP_BLK, 1), lambda i: (i, 0)),
            pl.BlockSpec((EMB, EMB), lambda i: (0, 0)),
            pl.BlockSpec((3, EMB), lambda i: (0, 0)),
            pl.BlockSpec((1, EMB), lambda i: (0, 0)),
            pl.BlockSpec((EMB, EMB), lambda i: (0, 0)),
            pl.BlockSpec((1, EMB), lambda i: (0, 0)),
        ],
        out_specs=pl.BlockSpec((MLP_BLK, EMB), lambda i: (i, 0)),
        out_shape=jax.ShapeDtypeStruct((BATCH, EMB), jnp.float32),
    )(stats, emb2, idx, age, msin, mcos, w1a, w1b, b1, w2, b2)


def kernel(customer_id, age, month_sin, month_cos, table, age_mean, age_var,
           W1, b1, W2, b2):
    idx = customer_id.astype(jnp.int32)
    pidx = jnp.left_shift(jnp.right_shift(idx, 13), 12) + jnp.bitwise_and(idx, HALF - 1)
    pairs = _tc_pair_relayout(table.T)
    emb2 = _sc_gather(pairs, pidx)
    stats = jnp.stack([age_mean, age_var]).reshape(1, 2)
    return _tc_mlp(
        emb2,
        idx.reshape(-1, 1),
        age.reshape(-1, 1),
        month_sin.reshape(-1, 1),
        month_cos.reshape(-1, 1),
        stats,
        W1[:EMB],
        W1[EMB:],
        b1.reshape(1, EMB),
        W2,
        b2.reshape(1, EMB),
    )


# speedup vs baseline: 17.7906x; 1.1048x over previous
import jax
import jax.numpy as jnp
from jax import lax
from jax.experimental import pallas as pl
from jax.experimental.pallas import tpu as pltpu
from jax.experimental.pallas import tpu_sc as plsc

BATCH = 16384
EMB = 64
VROWS = 1000001
NC = 2
NS = 16
NW = NC * NS
B_PER_W = BATCH // NW

TR_BLK = 16384
TR_STEPS = pl.cdiv(VROWS, TR_BLK)
PAIR_ROWS = TR_STEPS * (TR_BLK // 2)


HALF = TR_BLK // 2
TR_SHIFT = TR_BLK.bit_length() - 1


def _pair_body(src_ref, dst_ref):
    dst_ref[:, :EMB] = src_ref[:, :HALF].T
    dst_ref[:, EMB:] = src_ref[:, HALF:].T


def _tc_pair_relayout(tableT):
    return pl.pallas_call(
        _pair_body,
        grid=(TR_STEPS,),
        in_specs=[pl.BlockSpec((EMB, TR_BLK), lambda i: (0, i))],
        out_specs=pl.BlockSpec((TR_BLK // 2, 2 * EMB), lambda i: (i, 0)),
        out_shape=jax.ShapeDtypeStruct((PAIR_ROWS, 2 * EMB), jnp.float32),
    )(tableT)


def _gather_body(pairs_hbm, pidx_hbm, out_hbm, idx_v, rows_v, sem):
    wid = lax.axis_index("s") * NC + lax.axis_index("c")
    base = wid * B_PER_W
    pltpu.sync_copy(pidx_hbm.at[pl.ds(base, B_PER_W)], idx_v)
    pltpu.async_copy(pairs_hbm.at[idx_v], rows_v, sem).wait()
    pltpu.sync_copy(rows_v, out_hbm.at[pl.ds(base, B_PER_W)])


def _sc_gather(pairs, pidx):
    mesh = plsc.VectorSubcoreMesh(core_axis_name="c", subcore_axis_name="s")
    k = pl.kernel(
        _gather_body,
        mesh=mesh,
        out_type=jax.ShapeDtypeStruct((BATCH, 2 * EMB), jnp.float32),
        scratch_types=[
            pltpu.VMEM((B_PER_W,), jnp.int32),
            pltpu.VMEM((B_PER_W, 2 * EMB), jnp.float32),
            pltpu.SemaphoreType.DMA,
        ],
        compiler_params=pltpu.CompilerParams(use_tc_tiling_on_sc=False),
    )
    return k(pairs, pidx)


MLP_BLK = 2048


def _mlp_body(stats_ref, emb2_ref, idx_ref, age_ref, msin_ref, mcos_ref,
              w1a_ref, w1b_ref, b1_ref, w2_ref, b2_ref, out_ref):
    mean = stats_ref[0, 0]
    inv = lax.rsqrt(stats_ref[0, 1] + 1e-7)
    emb2 = emb2_ref[...]
    upper = lax.bitwise_and(
        lax.shift_right_logical(idx_ref[...], TR_SHIFT - 1), 1
    ) == 1
    emb = jnp.where(upper, emb2[:, EMB:], emb2[:, :EMB])
    h = jnp.dot(emb, w1a_ref[...], preferred_element_type=jnp.float32)
    h += ((age_ref[...] - mean) * inv) * w1b_ref[0:1, :]
    h += msin_ref[...] * w1b_ref[1:2, :]
    h += mcos_ref[...] * w1b_ref[2:3, :]
    h = jnp.maximum(h + b1_ref[...], 0.0)
    out_ref[...] = jnp.dot(h, w2_ref[...], preferred_element_type=jnp.float32) + b2_ref[...]


def _tc_mlp(emb2, idx, age, msin, mcos, stats, w1a, w1b, b1, w2, b2):
    grid = (BATCH // MLP_BLK,)
    return pl.pallas_call(
        _mlp_body,
        grid=grid,
        in_specs=[
            pl.BlockSpec(memory_space=pltpu.SMEM),
            pl.BlockSpec((MLP_BLK, 2 * EMB), lambda i: (i, 0)),
            pl.BlockSpec((MLP_BLK, 1), lambda i: (i, 0)),
            pl.BlockSpec((MLP_BLK, 1), lambda i: (i, 0)),
            pl.BlockSpec((MLP_BLK, 1), lambda i: (i, 0)),
            pl.BlockSpec((MLP_BLK, 1), lambda i: (i, 0)),
            pl.BlockSpec((EMB, EMB), lambda i: (0, 0)),
            pl.BlockSpec((3, EMB), lambda i: (0, 0)),
            pl.BlockSpec((1, EMB), lambda i: (0, 0)),
            pl.BlockSpec((EMB, EMB), lambda i: (0, 0)),
            pl.BlockSpec((1, EMB), lambda i: (0, 0)),
        ],
        out_specs=pl.BlockSpec((MLP_BLK, EMB), lambda i: (i, 0)),
        out_shape=jax.ShapeDtypeStruct((BATCH, EMB), jnp.float32),
    )(stats, emb2, idx, age, msin, mcos, w1a, w1b, b1, w2, b2)


def kernel(customer_id, age, month_sin, month_cos, table, age_mean, age_var,
           W1, b1, W2, b2):
    idx = customer_id.astype(jnp.int32)
    pidx = jnp.left_shift(jnp.right_shift(idx, TR_SHIFT), TR_SHIFT - 1) \
        + jnp.bitwise_and(idx, HALF - 1)
    pairs = _tc_pair_relayout(table.T)
    emb2 = _sc_gather(pairs, pidx)
    stats = jnp.stack([age_mean, age_var]).reshape(1, 2)
    return _tc_mlp(
        emb2,
        idx.reshape(-1, 1),
        age.reshape(-1, 1),
        month_sin.reshape(-1, 1),
        month_cos.reshape(-1, 1),
        stats,
        W1[:EMB],
        W1[EMB:],
        b1.reshape(1, EMB),
        W2,
        b2.reshape(1, EMB),
    )


# speedup vs baseline: 18.2931x; 1.0282x over previous
import jax
import jax.numpy as jnp
from jax import lax
from jax.experimental import pallas as pl
from jax.experimental.pallas import tpu as pltpu
from jax.experimental.pallas import tpu_sc as plsc

BATCH = 16384
EMB = 64
VROWS = 1000001
NC = 2
NS = 16
NW = NC * NS
B_PER_W = BATCH // NW

TR_BLK = 32768
TR_STEPS = pl.cdiv(VROWS, TR_BLK)
PAIR_ROWS = TR_STEPS * (TR_BLK // 2)


HALF = TR_BLK // 2
TR_SHIFT = TR_BLK.bit_length() - 1


def _pair_body(src_ref, dst_ref):
    dst_ref[:, :EMB] = src_ref[:, :HALF].T
    dst_ref[:, EMB:] = src_ref[:, HALF:].T


def _tc_pair_relayout(tableT):
    return pl.pallas_call(
        _pair_body,
        grid=(TR_STEPS,),
        in_specs=[pl.BlockSpec((EMB, TR_BLK), lambda i: (0, i))],
        out_specs=pl.BlockSpec((TR_BLK // 2, 2 * EMB), lambda i: (i, 0)),
        out_shape=jax.ShapeDtypeStruct((PAIR_ROWS, 2 * EMB), jnp.float32),
    )(tableT)


def _gather_body(pairs_hbm, pidx_hbm, out_hbm, idx_v, rows_v, sem):
    wid = lax.axis_index("s") * NC + lax.axis_index("c")
    base = wid * B_PER_W
    pltpu.sync_copy(pidx_hbm.at[pl.ds(base, B_PER_W)], idx_v)
    pltpu.async_copy(pairs_hbm.at[idx_v], rows_v, sem).wait()
    pltpu.sync_copy(rows_v, out_hbm.at[pl.ds(base, B_PER_W)])


def _sc_gather(pairs, pidx):
    mesh = plsc.VectorSubcoreMesh(core_axis_name="c", subcore_axis_name="s")
    k = pl.kernel(
        _gather_body,
        mesh=mesh,
        out_type=jax.ShapeDtypeStruct((BATCH, 2 * EMB), jnp.float32),
        scratch_types=[
            pltpu.VMEM((B_PER_W,), jnp.int32),
            pltpu.VMEM((B_PER_W, 2 * EMB), jnp.float32),
            pltpu.SemaphoreType.DMA,
        ],
        compiler_params=pltpu.CompilerParams(use_tc_tiling_on_sc=False),
    )
    return k(pairs, pidx)


MLP_BLK = 4096


def _mlp_body(stats_ref, emb2_ref, idx_ref, age_ref, msin_ref, mcos_ref,
              w1a_ref, w1b_ref, b1_ref, w2_ref, b2_ref, out_ref):
    mean = stats_ref[0, 0]
    inv = lax.rsqrt(stats_ref[0, 1] + 1e-7)
    emb2 = emb2_ref[...]
    upper = lax.bitwise_and(
        lax.shift_right_logical(idx_ref[...], TR_SHIFT - 1), 1
    ) == 1
    emb = jnp.where(upper, emb2[:, EMB:], emb2[:, :EMB])
    h = jnp.dot(emb, w1a_ref[...], preferred_element_type=jnp.float32)
    h += ((age_ref[...] - mean) * inv) * w1b_ref[0:1, :]
    h += msin_ref[...] * w1b_ref[1:2, :]
    h += mcos_ref[...] * w1b_ref[2:3, :]
    h = jnp.maximum(h + b1_ref[...], 0.0)
    out_ref[...] = lax.dot_general(
        w2_ref[...], h, (((0,), (1,)), ((), ())),
        preferred_element_type=jnp.float32,
    ) + b2_ref[...]


def _tc_mlp(emb2, idx, age, msin, mcos, stats, w1a, w1b, b1, w2, b2):
    grid = (BATCH // MLP_BLK,)
    return pl.pallas_call(
        _mlp_body,
        grid=grid,
        in_specs=[
            pl.BlockSpec(memory_space=pltpu.SMEM),
            pl.BlockSpec((MLP_BLK, 2 * EMB), lambda i: (i, 0)),
            pl.BlockSpec((MLP_BLK, 1), lambda i: (i, 0)),
            pl.BlockSpec((MLP_BLK, 1), lambda i: (i, 0)),
            pl.BlockSpec((MLP_BLK, 1), lambda i: (i, 0)),
            pl.BlockSpec((MLP_BLK, 1), lambda i: (i, 0)),
            pl.BlockSpec((EMB, EMB), lambda i: (0, 0)),
            pl.BlockSpec((3, EMB), lambda i: (0, 0)),
            pl.BlockSpec((1, EMB), lambda i: (0, 0)),
            pl.BlockSpec((EMB, EMB), lambda i: (0, 0)),
            pl.BlockSpec((EMB, 1), lambda i: (0, 0)),
        ],
        out_specs=pl.BlockSpec((EMB, MLP_BLK), lambda i: (0, i)),
        out_shape=jax.ShapeDtypeStruct((EMB, BATCH), jnp.float32),
    )(stats, emb2, idx, age, msin, mcos, w1a, w1b, b1, w2, b2)


def kernel(customer_id, age, month_sin, month_cos, table, age_mean, age_var,
           W1, b1, W2, b2):
    idx = customer_id.astype(jnp.int32)
    pidx = jnp.left_shift(jnp.right_shift(idx, TR_SHIFT), TR_SHIFT - 1) \
        + jnp.bitwise_and(idx, HALF - 1)
    pairs = _tc_pair_relayout(table.T)
    emb2 = _sc_gather(pairs, pidx)
    stats = jnp.stack([age_mean, age_var]).reshape(1, 2)
    outT = _tc_mlp(
        emb2,
        idx.reshape(-1, 1),
        age.reshape(-1, 1),
        month_sin.reshape(-1, 1),
        month_cos.reshape(-1, 1),
        stats,
        W1[:EMB],
        W1[EMB:],
        b1.reshape(1, EMB),
        W2,
        b2.reshape(EMB, 1),
    )
    return outT.T
